# baseline (speedup 1.0000x reference)
import functools

import jax
import jax.numpy as jnp
from jax.experimental import pallas as pl
from jax.experimental.pallas import tpu as pltpu

_NBUF = 3


def _make_tc_add(R, S, D, CH):
    n_tiles = R // CH

    def body(x_hbm, pos_hbm, out_hbm, posb, xbuf, psem, lsem, ssem):
        pos_cp = pltpu.async_copy(pos_hbm, posb, psem)

        def start_load(t):
            k = t % _NBUF
            return pltpu.async_copy(
                x_hbm.at[pl.ds(t * CH, CH)], xbuf.at[k], lsem.at[k])

        def start_store(t):
            k = t % _NBUF
            return pltpu.async_copy(
                xbuf.at[k], out_hbm.at[pl.ds(t * CH, CH)], ssem.at[k])

        loads = {t: start_load(t) for t in range(min(_NBUF, n_tiles))}
        stores = {}
        unretired = set()
        pos_cp.wait()

        for t in range(n_tiles):
            pt, nt = t - 2, t - 2 + _NBUF
            if pt >= 0 and nt < n_tiles:
                stores[pt].wait()
                unretired.discard(pt)
                loads[nt] = start_load(nt)
            k = t % _NBUF
            loads[t].wait()
            pbase = (t * CH) % S
            xbuf[k] = xbuf[k] + posb[pl.ds(pbase, CH), :]
            stores[t] = start_store(t)
            unretired.add(t)

        for t in sorted(unretired):
            stores[t].wait()

    return pl.pallas_call(
        body,
        grid=(),
        in_specs=[
            pl.BlockSpec(memory_space=pl.ANY),
            pl.BlockSpec(memory_space=pl.ANY),
        ],
        out_specs=pl.BlockSpec(memory_space=pl.ANY),
        out_shape=jax.ShapeDtypeStruct((R, D), jnp.float32),
        scratch_shapes=[
            pltpu.VMEM((S, D), jnp.float32),
            pltpu.VMEM((_NBUF, CH, D), jnp.float32),
            pltpu.SemaphoreType.DMA,
            pltpu.SemaphoreType.DMA((_NBUF,)),
            pltpu.SemaphoreType.DMA((_NBUF,)),
        ],
    )


def kernel(x, pos_table):
    B, S, D = x.shape
    out = _make_tc_add(B * S, S, D, CH=2048)(
        x.reshape(B * S, D), pos_table[:S])
    return out.reshape(B, S, D)


# speedup vs baseline: 1.0412x; 1.0412x over previous
import jax
import jax.numpy as jnp
from jax.experimental import pallas as pl


def _add_body(x_ref, pos_ref, o_ref):
    o_ref[...] = x_ref[...] + pos_ref[...]


def kernel(x, pos_table):
    B, S, D = x.shape
    return pl.pallas_call(
        _add_body,
        grid=(B,),
        in_specs=[
            pl.BlockSpec((1, S, D), lambda j: (j, 0, 0)),
            pl.BlockSpec((S, D), lambda j: (0, 0)),
        ],
        out_specs=pl.BlockSpec((1, S, D), lambda j: (j, 0, 0)),
        out_shape=jax.ShapeDtypeStruct((B, S, D), x.dtype),
    )(x, pos_table[:S])
